# Initial kernel scaffold; baseline (speedup 1.0000x reference)
import jax
import jax.numpy as jnp
from jax.experimental import pallas as pl


def kernel(x, edge_index, W_l, b_l, W_r, W_lin, b_lin):
    raise NotImplementedError("write your pallas kernel here")


# speedup vs baseline: 7.9597x; 7.9597x over previous
import functools

import jax
import jax.numpy as jnp
from jax import lax
from jax.experimental import pallas as pl
from jax.experimental.pallas import tpu as pltpu
from jax.experimental.pallas import tpu_sc as plsc

N, E, D, H, O = 10000, 320000, 128, 64, 64
NP = 10240
NC, NS = 2, 16
NW = NC * NS
NB = 80
BATCH = 128
EPT = NB * BATCH
EP = NW * EPT
ROWS_PT = NP // NS


CW = 8


def _sc_body(y_hbm, src_hbm, dst_hbm, zrows_hbm, zcnt_hbm, ones_hbm,
             sums_hbm, cnt_hbm,
             src_v, dst_v, buf0, buf1, ones_v, acc, cacc, sem0, sem1):
  cid = lax.axis_index("c")
  sid = lax.axis_index("s")
  wid = sid * NC + cid

  pltpu.sync_copy(zrows_hbm, acc.at[pl.ds(sid * ROWS_PT, ROWS_PT)])
  pltpu.sync_copy(zcnt_hbm, cacc.at[pl.ds(sid * ROWS_PT, ROWS_PT)])
  pltpu.sync_copy(ones_hbm, ones_v)
  pltpu.sync_copy(src_hbm.at[wid], src_v)
  pltpu.sync_copy(dst_hbm.at[wid], dst_v)
  plsc.subcore_barrier()

  pltpu.async_copy(y_hbm.at[src_v.at[0]], buf0, sem0)

  def _pair(i, carry):
    b0 = 2 * i
    b1 = 2 * i + 1
    pltpu.async_copy(y_hbm.at[src_v.at[b1]], buf1, sem1)
    pltpu.make_async_copy(y_hbm.at[src_v.at[b0]], buf0, sem0).wait()
    pltpu.sync_copy(buf0, acc.at[dst_v.at[b0]], add=True)
    pltpu.sync_copy(ones_v, cacc.at[dst_v.at[b0]], add=True)

    @pl.when(i < NB // 2 - 1)
    def _():
      pltpu.async_copy(y_hbm.at[src_v.at[b0 + 2]], buf0, sem0)

    pltpu.make_async_copy(y_hbm.at[src_v.at[b1]], buf1, sem1).wait()
    pltpu.sync_copy(buf1, acc.at[dst_v.at[b1]], add=True)
    pltpu.sync_copy(ones_v, cacc.at[dst_v.at[b1]], add=True)
    return carry

  lax.fori_loop(0, NB // 2, _pair, 0)
  plsc.subcore_barrier()

  pltpu.sync_copy(acc.at[pl.ds(sid * ROWS_PT, ROWS_PT)],
                  sums_hbm.at[cid, pl.ds(sid * ROWS_PT, ROWS_PT)])
  pltpu.sync_copy(cacc.at[pl.ds(sid * ROWS_PT, ROWS_PT)],
                  cnt_hbm.at[cid, pl.ds(sid * ROWS_PT, ROWS_PT)])


_sc_segment_mean_parts = functools.partial(
    pl.kernel,
    out_type=[
        jax.ShapeDtypeStruct((NC, NP, H), jnp.float32),
        jax.ShapeDtypeStruct((NC, NP, CW), jnp.float32),
    ],
    mesh=plsc.VectorSubcoreMesh(core_axis_name="c", subcore_axis_name="s"),
    compiler_params=pltpu.CompilerParams(use_tc_tiling_on_sc=False),
    scratch_types=[
        pltpu.VMEM((NB, BATCH), jnp.int32),
        pltpu.VMEM((NB, BATCH), jnp.int32),
        pltpu.VMEM((BATCH, H), jnp.float32),
        pltpu.VMEM((BATCH, H), jnp.float32),
        pltpu.VMEM((BATCH, CW), jnp.float32),
        pltpu.VMEM_SHARED((NP, H), jnp.float32),
        pltpu.VMEM_SHARED((NP, CW), jnp.float32),
        pltpu.SemaphoreType.DMA,
        pltpu.SemaphoreType.DMA,
    ],
)(_sc_body)


def _mm_body(x_ref, wl_ref, wr_ref, y_ref, z_ref):
  xb = x_ref[...]
  y_ref[...] = jnp.dot(xb, wl_ref[...], preferred_element_type=jnp.float32)
  z_ref[...] = jnp.dot(xb, wr_ref[...], preferred_element_type=jnp.float32)


def _tc_in_proj(x_pad, W_l, W_r):
  blk = NP // 8
  return pl.pallas_call(
      _mm_body,
      grid=(8,),
      in_specs=[
          pl.BlockSpec((blk, D), lambda i: (i, 0)),
          pl.BlockSpec((D, H), lambda i: (0, 0)),
          pl.BlockSpec((D, H), lambda i: (0, 0)),
      ],
      out_specs=[
          pl.BlockSpec((blk, H), lambda i: (i, 0)),
          pl.BlockSpec((blk, H), lambda i: (i, 0)),
      ],
      out_shape=[
          jax.ShapeDtypeStruct((NP, H), jnp.float32),
          jax.ShapeDtypeStruct((NP, H), jnp.float32),
      ],
      compiler_params=pltpu.CompilerParams(
          dimension_semantics=("parallel",)),
  )(x_pad, W_l, W_r)


def _out_body(sums_ref, cnt_ref, z_ref, bl_ref, wlin_ref, blin_ref, o_ref):
  s = sums_ref[0] + sums_ref[1]
  c = (cnt_ref[0] + cnt_ref[1])[:, 0:1]
  mean = s / jnp.maximum(c, 1.0)
  h = mean + bl_ref[...] + z_ref[...]
  h = jnp.where(h > 0.0, h, jnp.exp(jnp.minimum(h, 0.0)) - 1.0)
  o_ref[...] = (jnp.dot(h, wlin_ref[...], preferred_element_type=jnp.float32)
                + blin_ref[...])


def _tc_out_proj(sums, cnts, z, b_l, W_lin, b_lin):
  blk = NP // 8
  return pl.pallas_call(
      _out_body,
      grid=(8,),
      in_specs=[
          pl.BlockSpec((NC, blk, H), lambda i: (0, i, 0)),
          pl.BlockSpec((NC, blk, CW), lambda i: (0, i, 0)),
          pl.BlockSpec((blk, H), lambda i: (i, 0)),
          pl.BlockSpec((1, H), lambda i: (0, 0)),
          pl.BlockSpec((H, O), lambda i: (0, 0)),
          pl.BlockSpec((1, O), lambda i: (0, 0)),
      ],
      out_specs=pl.BlockSpec((blk, O), lambda i: (i, 0)),
      out_shape=jax.ShapeDtypeStruct((NP, O), jnp.float32),
      compiler_params=pltpu.CompilerParams(
          dimension_semantics=("parallel",)),
  )(sums, cnts, z, b_l.reshape(1, H), W_lin, b_lin.reshape(1, O))


def kernel(x, edge_index, W_l, b_l, W_r, W_lin, b_lin):
  x_pad = jnp.zeros((NP, D), jnp.float32).at[:N].set(x)
  y, z = _tc_in_proj(x_pad, W_l, W_r)

  pad_e = EP - E
  src_p = jnp.concatenate(
      [edge_index[0], jnp.zeros((pad_e,), jnp.int32)]).reshape(NW, NB, BATCH)
  dst_p = jnp.concatenate(
      [edge_index[1], jnp.full((pad_e,), N, jnp.int32)]).reshape(NW, NB, BATCH)

  zrows = jnp.zeros((ROWS_PT, H), jnp.float32)
  zcnt = jnp.zeros((ROWS_PT, CW), jnp.float32)
  ones_rows = jnp.zeros((BATCH, CW), jnp.float32).at[:, 0].set(1.0)
  sums, cnts = _sc_segment_mean_parts(y, src_p, dst_p, zrows, zcnt, ones_rows)

  out = _tc_out_proj(sums, cnts, z, b_l, W_lin, b_lin)
  return out[:N]
